# baseline (speedup 1.0000x reference)
import functools

import jax
import jax.numpy as jnp
from jax import lax
from jax.experimental import pallas as pl
from jax.experimental.pallas import tpu as pltpu
from jax.experimental.pallas import tpu_sc as plsc

_PAD = 1
_C = 16
_R = 4


def _sc_body(tok_hbm, w_hbm, out_hbm, tokv, buf, idxq, insems, outsems):
    bsz, seq_len, _ = out_hbm.shape
    n_workers = 32
    t_per_w = seq_len // n_workers
    n_chunks = t_per_w // _C
    total = n_chunks * bsz
    wid = lax.axis_index("s") * 2 + lax.axis_index("c")
    tbase = wid * t_per_w

    for b in range(bsz):
        pltpu.sync_copy(
            tok_hbm.at[pl.ds(b * seq_len + tbase, t_per_w)], tokv.at[b]
        )

    def item(k):
        g, b = divmod(k, bsz)
        return g, b, k % _R

    def start(k):
        g, b, s = item(k)
        v = tokv[b, pl.ds(g * _C, _C)]
        pos = jax.lax.broadcasted_iota(jnp.int32, (_C,), 0) + (
            tbase + g * _C + 2
        )
        idxq[s, ...] = jnp.where(v == _PAD, _PAD, pos)
        return pltpu.async_copy(w_hbm.at[idxq.at[s]], buf.at[s], insems.at[s])

    def fire_out(k):
        g, b, s = item(k)
        return pltpu.async_copy(
            buf.at[s], out_hbm.at[b, pl.ds(tbase + g * _C, _C)], outsems.at[s]
        )

    def drain_out(k):
        g, b, s = item(k)
        pltpu.make_async_copy(
            buf.at[s], out_hbm.at[b, pl.ds(tbase + g * _C, _C)], outsems.at[s]
        ).wait()

    in_handles = {k: start(k) for k in range(min(2, total))}
    for k in range(total):
        if k >= 2:
            drain_out(k - 2)
        if k + 2 < total:
            in_handles[k + 2] = start(k + 2)
        in_handles.pop(k).wait()
        fire_out(k)
    for k in range(max(total - 2, 0), total):
        drain_out(k)


def kernel(input, weights):
    bsz, seq_len = input.shape
    dim = weights.shape[1]
    mesh = plsc.VectorSubcoreMesh(core_axis_name="c", subcore_axis_name="s")
    t_per_w = seq_len // 32
    k = functools.partial(
        pl.kernel,
        mesh=mesh,
        out_type=jax.ShapeDtypeStruct((bsz, seq_len, dim), weights.dtype),
        scratch_types=[
            pltpu.VMEM((bsz, t_per_w), jnp.int32),
            pltpu.VMEM((_R, _C, dim), jnp.float32),
            pltpu.VMEM((_R, _C), jnp.int32),
            pltpu.SemaphoreType.DMA((_R,)),
            pltpu.SemaphoreType.DMA((_R,)),
        ],
    )(_sc_body)
    return k(input.reshape(-1), weights)
